# Initial kernel scaffold; baseline (speedup 1.0000x reference)
import jax
import jax.numpy as jnp
from jax.experimental import pallas as pl


def kernel(x, edge_index, W1_0, W2_0, b_0, W1_1, W2_1, b_1, W1_2, W2_2, b_2, g_0, be_0, g_1, be_1):
    raise NotImplementedError("write your pallas kernel here")


# speedup vs baseline: 4.0877x; 4.0877x over previous
import functools

import jax
import jax.numpy as jnp
from jax import lax
from jax.experimental import pallas as pl
from jax.experimental.pallas import tpu as pltpu
from jax.experimental.pallas import tpu_sc as plsc

N = 10000
E = 320000
D = 128

NC = 2
NS = 16
NW = NC * NS
EPW = E // NW
C = 80
NCHUNK = EPW // C
NP = 10240
ZR = NP // NS

_sc_mesh = plsc.VectorSubcoreMesh(
    core_axis_name="c", subcore_axis_name="s", num_cores=NC, num_subcores=NS)


@functools.partial(
    pl.kernel,
    out_type=jax.ShapeDtypeStruct((NC, NP, D), jnp.float32),
    mesh=_sc_mesh,
    scratch_types=[
        pltpu.VMEM((C,), jnp.int32),
        pltpu.VMEM((C,), jnp.int32),
        pltpu.VMEM((C, D), jnp.float32),
        pltpu.VMEM_SHARED((NP, D), jnp.float32),
        pltpu.SemaphoreType.DMA,
    ],
)
def _sc_agg(h_hbm, src_hbm, dst_hbm, zero_hbm, out_hbm, sidx, didx, rows, acc, sem):
    cid = lax.axis_index("c")
    sid = lax.axis_index("s")
    wid = cid * NS + sid
    pltpu.sync_copy(zero_hbm.at[pl.ds(sid * ZR, ZR)], acc.at[pl.ds(sid * ZR, ZR)])
    plsc.subcore_barrier()

    def step(i, carry):
        e0 = wid * EPW + i * C
        pltpu.sync_copy(src_hbm.at[pl.ds(e0, C)], sidx)
        pltpu.sync_copy(dst_hbm.at[pl.ds(e0, C)], didx)
        pltpu.async_copy(h_hbm.at[sidx], rows, sem).wait()
        pltpu.sync_copy(rows, acc.at[didx], add=True)
        return carry

    lax.fori_loop(0, NCHUNK, step, 0)
    plsc.subcore_barrier()
    pltpu.sync_copy(acc.at[pl.ds(sid * ZR, ZR)],
                    out_hbm.at[cid, pl.ds(sid * ZR, ZR)])


_RB = 1000


def _combine_body(has_bn, h, accp, cntp, w1, w2, b, g, be, out):
    s = accp[0] + accp[1]
    c = cntp[0, :, 0:1] + cntp[1, :, 0:1]
    hn = s * (1.0 / jnp.maximum(c, 1.0))
    h2 = (lax.dot_general(h[...], w1[...], (((1,), (1,)), ((), ())),
                          preferred_element_type=jnp.float32)
          + lax.dot_general(hn, w2[...], (((1,), (1,)), ((), ())),
                            preferred_element_type=jnp.float32)
          + b[...])
    nrm = jnp.maximum(jnp.sqrt(jnp.sum(h2 * h2, axis=1, keepdims=True)), 1e-12)
    y = h2 / nrm
    if has_bn:
        y = y * (g[...] / jnp.sqrt(1.0 + 1e-5)) + be[...]
        y = jnp.maximum(y, 0.0)
    out[...] = y


def _combine(h, acc, cnt, w1, w2, b, g, be, has_bn):
    mat = pl.BlockSpec((D, D), lambda i: (0, 0))
    vec = pl.BlockSpec((1, D), lambda i: (0, 0))
    return pl.pallas_call(
        functools.partial(_combine_body, has_bn),
        grid=(N // _RB,),
        in_specs=[
            pl.BlockSpec((_RB, D), lambda i: (i, 0)),
            pl.BlockSpec((NC, _RB, D), lambda i: (0, i, 0)),
            pl.BlockSpec((NC, _RB, 8), lambda i: (0, i, 0)),
            mat, mat, vec, vec, vec,
        ],
        out_specs=pl.BlockSpec((_RB, D), lambda i: (i, 0)),
        out_shape=jax.ShapeDtypeStruct((N, D), jnp.float32),
    )(h, acc, cnt, w1, w2, b[None, :], g[None, :], be[None, :])


def kernel(x, edge_index, W1_0, W2_0, b_0, W1_1, W2_1, b_1, W1_2, W2_2, b_2,
           g_0, be_0, g_1, be_1):
    ei = edge_index.astype(jnp.int32)
    src = ei[0]
    dst = ei[1]
    zero_nd = jnp.zeros((NP, D), jnp.float32)
    ones_nd = jnp.ones((N, D), jnp.float32)

    cnt = _sc_agg(ones_nd, src, dst, zero_nd)[:, :, :8]

    h = x
    layers = [
        (W1_0, W2_0, b_0, g_0, be_0, True),
        (W1_1, W2_1, b_1, g_1, be_1, True),
        (W1_2, W2_2, b_2, g_1, be_1, False),
    ]
    for w1, w2, b, g, be, has_bn in layers:
        acc = _sc_agg(h, src, dst, zero_nd)
        h = _combine(h, acc, cnt, w1, w2, b, g, be, has_bn)
    return h
